# Initial kernel scaffold; baseline (speedup 1.0000x reference)
import jax
import jax.numpy as jnp
from jax.experimental import pallas as pl


def kernel(orbital_features, atomic_table, orbital_table, m_table, W, b):
    raise NotImplementedError("write your pallas kernel here")


# speedup vs baseline: 5.6138x; 5.6138x over previous
import functools

import jax
import jax.numpy as jnp
from jax import lax
from jax.experimental import pallas as pl
from jax.experimental.pallas import tpu as pltpu
from jax.experimental.pallas import tpu_sc as plsc

N_ROWS = 2_000_000
EMB = 32
N_TBL = 21 * 4 * 7
NW = 32
ROWS_PER_W = N_ROWS // NW
C = 500
CP = 512
CHUNKS = ROWS_PER_W // C
G_PER_CHUNK = CP // 16


def _sc_body(feat_hbm, tbl_hbm, wc_hbm, out_hbm, feats_v, tbl_v, wc_v, out_v):
    wid = lax.axis_index("s") * 2 + lax.axis_index("c")
    pltpu.sync_copy(tbl_hbm, tbl_v)
    pltpu.sync_copy(wc_hbm, wc_v)
    w_lo = [wc_v[k, pl.ds(0, 16)] for k in range(5)]
    w_hi = [wc_v[k, pl.ds(16, 16)] for k in range(5)]
    lanes8 = lax.iota(jnp.int32, 16) * 8
    base0 = wid * (ROWS_PER_W * 8)

    def chunk(ci, carry):
        base = base0 + ci * (C * 8)
        pltpu.sync_copy(feat_hbm.at[pl.ds(base, C * 8)],
                        feats_v.at[pl.ds(0, C * 8)])

        def group(g, cr):
            e0 = g * 128 + lanes8
            af = plsc.load_gather(feats_v, [e0])
            of = plsc.load_gather(feats_v, [e0 + 1])
            mf = plsc.load_gather(feats_v, [e0 + 2])
            tv = (af.astype(jnp.int32) * 28 + of.astype(jnp.int32) * 7
                  + mf.astype(jnp.int32) + 3)
            tv = jnp.clip(tv, 0, N_TBL - 1)
            ck = [plsc.load_gather(feats_v, [e0 + 3 + k]) for k in range(5)]
            for j in range(16):
                t = tv[j]
                acc_lo = tbl_v[t, pl.ds(0, 16)]
                acc_hi = tbl_v[t, pl.ds(16, 16)]
                for k in range(5):
                    c = ck[k][j]
                    acc_lo = acc_lo + c * w_lo[k]
                    acc_hi = acc_hi + c * w_hi[k]
                r = (g * 16 + j) * EMB
                out_v[pl.ds(r, 16)] = acc_lo
                out_v[pl.ds(r + 16, 16)] = acc_hi
            return cr

        lax.fori_loop(0, G_PER_CHUNK, group, 0)
        pltpu.sync_copy(out_v.at[pl.ds(0, C * EMB)],
                        out_hbm.at[pl.ds(base0 * 4 + ci * (C * EMB), C * EMB)])
        return carry

    lax.fori_loop(0, CHUNKS, chunk, 0)


@jax.jit
def _sc_call(feats_flat, tbl, wc):
    mesh = plsc.VectorSubcoreMesh(core_axis_name="c", subcore_axis_name="s")
    f = pl.kernel(
        _sc_body,
        mesh=mesh,
        compiler_params=pltpu.CompilerParams(needs_layout_passes=False),
        out_type=jax.ShapeDtypeStruct((N_ROWS * EMB,), jnp.float32),
        scratch_types=[
            pltpu.VMEM((CP * 8,), jnp.float32),
            pltpu.VMEM((N_TBL, EMB), jnp.float32),
            pltpu.VMEM((5, EMB), jnp.float32),
            pltpu.VMEM((CP * EMB,), jnp.float32),
        ],
    )
    return f(feats_flat, tbl, wc)


def kernel(orbital_features, atomic_table, orbital_table, m_table, W, b):
    A2 = atomic_table @ W[:, 5:37].T
    O2 = orbital_table @ W[:, 37:53].T
    M2 = m_table @ W[:, 53:61].T
    T = (A2[:, None, None, :] + O2[None, :, None, :] + M2[None, None, :, :]
         + b).reshape(N_TBL, EMB).astype(jnp.float32)
    Wc = W[:, :5].T.astype(jnp.float32)
    feats_flat = orbital_features.reshape(-1).astype(jnp.float32)
    return _sc_call(feats_flat, T, Wc).reshape(N_ROWS, EMB)


# speedup vs baseline: 5.7613x; 1.0263x over previous
import functools

import jax
import jax.numpy as jnp
from jax import lax
from jax.experimental import pallas as pl
from jax.experimental.pallas import tpu as pltpu
from jax.experimental.pallas import tpu_sc as plsc

N_ROWS = 2_000_000
EMB = 32
N_TBL = 21 * 4 * 7
NW = 32
ROWS_PER_W = N_ROWS // NW
C = 500
CP = 512
CHUNKS = ROWS_PER_W // C
G_PER_CHUNK = CP // 16


def _sc_body(feat_hbm, tbl_hbm, wc_hbm, out_hbm, feats_v, tbl_v, wc_v, out_v):
    wid = lax.axis_index("s") * 2 + lax.axis_index("c")
    pltpu.sync_copy(tbl_hbm, tbl_v)
    pltpu.sync_copy(wc_hbm, wc_v)
    w_lo = [wc_v[k, pl.ds(0, 16)] for k in range(5)]
    w_hi = [wc_v[k, pl.ds(16, 16)] for k in range(5)]
    lanes = lax.iota(jnp.int32, 16)
    lanes8 = lanes * 8
    base0 = wid * (ROWS_PER_W * 8)

    def chunk(ci, carry):
        base = base0 + ci * (C * 8)
        pltpu.sync_copy(feat_hbm.at[pl.ds(base, C * 8)],
                        feats_v.at[pl.ds(0, C * 8)])

        def group(g, cr):
            e0 = g * 128 + lanes8
            af = plsc.load_gather(feats_v, [e0])
            of = plsc.load_gather(feats_v, [e0 + 1])
            mf = plsc.load_gather(feats_v, [e0 + 2])
            tv = (af.astype(jnp.int32) * 28 + of.astype(jnp.int32) * 7
                  + mf.astype(jnp.int32) + 3)
            tv = jnp.clip(tv, 0, N_TBL - 1)
            tw = tv * EMB
            ck = [plsc.load_gather(feats_v, [e0 + 3 + k]) for k in range(5)]
            for j in range(16):
                idx_lo = tw[j] + lanes
                acc_lo = plsc.load_gather(tbl_v, [idx_lo])
                acc_hi = plsc.load_gather(tbl_v, [idx_lo + 16])
                for k in range(5):
                    c = ck[k][j]
                    acc_lo = acc_lo + c * w_lo[k]
                    acc_hi = acc_hi + c * w_hi[k]
                r = (g * 16 + j) * EMB
                out_v[pl.ds(r, 16)] = acc_lo
                out_v[pl.ds(r + 16, 16)] = acc_hi
            return cr

        lax.fori_loop(0, G_PER_CHUNK, group, 0)
        pltpu.sync_copy(out_v.at[pl.ds(0, C * EMB)],
                        out_hbm.at[pl.ds(base0 * 4 + ci * (C * EMB), C * EMB)])
        return carry

    lax.fori_loop(0, CHUNKS, chunk, 0)


@jax.jit
def _sc_call(feats_flat, tbl, wc):
    mesh = plsc.VectorSubcoreMesh(core_axis_name="c", subcore_axis_name="s")
    f = pl.kernel(
        _sc_body,
        mesh=mesh,
        compiler_params=pltpu.CompilerParams(needs_layout_passes=False),
        out_type=jax.ShapeDtypeStruct((N_ROWS * EMB,), jnp.float32),
        scratch_types=[
            pltpu.VMEM((CP * 8,), jnp.float32),
            pltpu.VMEM((N_TBL * EMB,), jnp.float32),
            pltpu.VMEM((5, EMB), jnp.float32),
            pltpu.VMEM((CP * EMB,), jnp.float32),
        ],
    )
    return f(feats_flat, tbl, wc)


def kernel(orbital_features, atomic_table, orbital_table, m_table, W, b):
    A2 = atomic_table @ W[:, 5:37].T
    O2 = orbital_table @ W[:, 37:53].T
    M2 = m_table @ W[:, 53:61].T
    T = (A2[:, None, None, :] + O2[None, :, None, :] + M2[None, None, :, :]
         + b).reshape(N_TBL * EMB).astype(jnp.float32)
    Wc = W[:, :5].T.astype(jnp.float32)
    feats_flat = orbital_features.reshape(-1).astype(jnp.float32)
    return _sc_call(feats_flat, T, Wc).reshape(N_ROWS, EMB)
